# Initial kernel scaffold; baseline (speedup 1.0000x reference)
import jax
import jax.numpy as jnp
from jax.experimental import pallas as pl


def kernel(x_1, x_2, edge_idx_1, edge_idx_2, edge_attr_1, edge_attr_2, params):
    raise NotImplementedError("write your pallas kernel here")


# speedup vs baseline: 59.5422x; 59.5422x over previous
import numpy as np
import jax
import jax.numpy as jnp
from jax.experimental import pallas as pl

D = 300
PD = 320
NEG = 0.01
_SCALE = float(1.0 / np.sqrt(float(D)))


def _lrelu(x):
    return jnp.where(x >= 0, x, NEG * x)


def _mm(a, b):
    return jax.lax.dot_general(a, b, (((1,), (0,)), ((), ())),
                               preferred_element_type=jnp.float32)


def _mm_t(a, b):
    return jax.lax.dot_general(a, b, (((1,), (1,)), ((), ())),
                               preferred_element_type=jnp.float32)


def _intra_body(x_ref, ea_ref, w_ref, b_ref, we_ref, be_ref,
                src_ref, dst_ref, o_ref):
    x = x_ref[...]
    n = x.shape[0]
    ne = ea_ref.shape[0]
    q = _mm(x, w_ref[0]) + b_ref[0:1, :]
    k = _mm(x, w_ref[1]) + b_ref[1:2, :]
    v = _mm(x, w_ref[2]) + b_ref[2:3, :]
    s = _mm(x, w_ref[3]) + b_ref[3:4, :]
    e = _mm(ea_ref[...], we_ref[...]) + be_ref[...]

    src = src_ref[...]
    dst = dst_ref[...]
    row_ids = jax.lax.broadcasted_iota(jnp.int32, (n, ne), 0)
    msrc = (row_ids == src).astype(jnp.float32)
    mdst_b = (row_ids == dst)

    sqk = _mm_t(q, k)
    rows = _mm(sqk, msrc) + _mm_t(q, e)
    alpha = jnp.sum(jnp.where(mdst_b, rows, 0.0), axis=0, keepdims=True) * _SCALE

    a_dense = jnp.where(mdst_b, alpha, -jnp.inf)
    amax = jnp.max(a_dense, axis=1, keepdims=True)
    amax = jnp.where(amax == -jnp.inf, 0.0, amax)
    p = jnp.exp(a_dense - amax)
    denom = jnp.sum(p, axis=1, keepdims=True)
    pn = p / (denom + 1e-16)

    c = _mm_t(pn, msrc)
    out = _mm(c, v) + _mm(pn, e) + s
    o_ref[...] = _lrelu(out)


def _cross_body(x1_ref, x2_ref,
                wt_ref, bt_ref, wet_ref, bet_ref,
                wg_ref, bg_ref, weg_ref, beg_ref,
                w1_ref, b1_ref, w2_ref, b2_ref, w3_ref, b3_ref, o_ref):
    x1 = x1_ref[...]
    x2 = x2_ref[...]

    def conv(xd, xs, w, b, we, be):
        ec = jnp.sum(we[...], axis=0, keepdims=True) + be[...]
        qd = _mm(xd, w[0]) + b[0:1, :]
        ks = _mm(xs, w[1]) + b[1:2, :] + ec
        vs = _mm(xs, w[2]) + b[2:3, :] + ec
        sd = _mm(xd, w[3]) + b[3:4, :]
        al = _mm_t(qd, ks) * _SCALE
        amax = jnp.max(al, axis=1, keepdims=True)
        p = jnp.exp(al - amax)
        denom = jnp.sum(p, axis=1, keepdims=True)
        pn = p / (denom + 1e-16)
        return _lrelu(_mm(pn, vs) + sd)

    x1n = conv(x1, x2, wt_ref, bt_ref, wet_ref, bet_ref)
    x2n = conv(x2, x1, wg_ref, bg_ref, weg_ref, beg_ref)

    p1 = jnp.mean(x1n, axis=0, keepdims=True)
    p2 = jnp.mean(x2n, axis=0, keepdims=True)
    xc = jnp.concatenate([p1, p2], axis=1)
    h = _lrelu(_mm(xc, w1_ref[...]) + b1_ref[...])
    h = _lrelu(_mm(h, w2_ref[...]) + b2_ref[...])
    o = _mm(h, w3_ref[...]) + b3_ref[...]
    o = 1.0 / (1.0 + jnp.exp(-o))

    o_ref[...] = jnp.zeros((8, 2 * PD), jnp.float32)
    o_ref[0:1, 0:PD] = p1
    o_ref[1:2, 0:PD] = p2
    o_ref[2:3, 0:128] = o


def _padw(w):
    return jnp.pad(w, ((0, PD - D), (0, PD - D)))


def _padb(b):
    return jnp.pad(b, (0, PD - D)).reshape(1, PD)


def _stack_conv(p):
    w = jnp.stack([_padw(p['Wq'][0]), _padw(p['Wk'][0]),
                   _padw(p['Wv'][0]), _padw(p['Ws'][0])])
    b = jnp.concatenate([_padb(p['bq'][0]), _padb(p['bk'][0]),
                         _padb(p['bv'][0]), _padb(p['bs'][0])], axis=0)
    return w, b, _padw(p['We'][0]), _padb(p['be'][0])


def kernel(x_1, x_2, edge_idx_1, edge_idx_2, edge_attr_1, edge_attr_2, params):
    n = x_1.shape[0]
    ne = edge_idx_1.shape[1]
    f32 = jnp.float32

    padx = lambda x: jnp.pad(x, ((0, 0), (0, PD - D)))
    wT, bT, weT, beT = _stack_conv(params['TSA'])
    wG, bG, weG, beG = _stack_conv(params['GSA'])
    wTC, bTC, weTC, beTC = _stack_conv(params['TCA'])
    wGC, bGC, weGC, beGC = _stack_conv(params['GCA'])

    src1 = edge_idx_1[0].astype(jnp.int32).reshape(1, ne)
    dst1 = edge_idx_1[1].astype(jnp.int32).reshape(1, ne)
    src2 = edge_idx_2[0].astype(jnp.int32).reshape(1, ne)
    dst2 = edge_idx_2[1].astype(jnp.int32).reshape(1, ne)

    intra = pl.pallas_call(
        _intra_body, out_shape=jax.ShapeDtypeStruct((n, PD), f32))
    x1p = intra(padx(x_1), padx(edge_attr_1), wT, bT, weT, beT, src1, dst1)
    x2p = intra(padx(x_2), padx(edge_attr_2), wG, bG, weG, beG, src2, dst2)

    m = params['mlp']
    w1p = jnp.concatenate([jnp.pad(m['W1'][:D], ((0, PD - D), (0, 0))),
                           jnp.pad(m['W1'][D:], ((0, PD - D), (0, 0)))], axis=0)
    b1p = m['b1'].reshape(1, -1)
    b2p = m['b2'].reshape(1, -1)
    w3p = jnp.pad(m['W3'], ((0, 0), (0, 127)))
    b3p = jnp.pad(m['b3'], (0, 127)).reshape(1, 128)

    packed = pl.pallas_call(
        _cross_body, out_shape=jax.ShapeDtypeStruct((8, 2 * PD), f32))(
        x1p, x2p, wTC, bTC, weTC, beTC, wGC, bGC, weGC, beGC,
        w1p, b1p, m['W2'], b2p, w3p, b3p)

    p1 = packed[0, :D]
    p2 = packed[1, :D]
    out = packed[2, :1]
    return (p1, p2, out)
